# baseline (speedup 1.0000x reference)
import functools

import jax
import jax.numpy as jnp
from jax import lax
from jax.experimental import pallas as pl
from jax.experimental.pallas import tpu as pltpu
from jax.experimental.pallas import tpu_sc as plsc

B = 16384
D = 64
NC = 2
NS = 16
NW = NC * NS
BPW = B // NW


def _sc_gather3(user_hbm, item_hbm, uid_hbm, iid_hbm, nid_hbm,
                u_out, p_out, n_out,
                uidx, iidx, nidx, sem):
    wid = lax.axis_index("s") * NC + lax.axis_index("c")
    base = wid * BPW
    pltpu.sync_copy(uid_hbm.at[wid], uidx)
    pltpu.sync_copy(iid_hbm.at[wid], iidx)
    pltpu.sync_copy(nid_hbm.at[wid], nidx)

    def body(g, _):
        i0 = g * 16
        uvec = uidx[pl.ds(i0, 16)]
        ivec = iidx[pl.ds(i0, 16)]
        nvec = nidx[pl.ds(i0, 16)]
        for j in range(16):
            dst = pl.ds(base + i0 + j, 1)
            pltpu.async_copy(user_hbm.at[pl.ds(uvec[j], 1)], u_out.at[dst], sem)
            pltpu.async_copy(item_hbm.at[pl.ds(ivec[j], 1)], p_out.at[dst], sem)
            pltpu.async_copy(item_hbm.at[pl.ds(nvec[j], 1)], n_out.at[dst], sem)
        return 0

    lax.fori_loop(0, BPW // 16, body, 0)
    osl = pl.ds(base, BPW)
    pltpu.make_async_copy(user_hbm.at[osl], u_out.at[osl], sem).wait()
    pltpu.make_async_copy(user_hbm.at[osl], p_out.at[osl], sem).wait()
    pltpu.make_async_copy(user_hbm.at[osl], n_out.at[osl], sem).wait()


_gather3 = functools.partial(
    pl.kernel,
    mesh=plsc.VectorSubcoreMesh(core_axis_name="c", subcore_axis_name="s"),
    out_type=[jax.ShapeDtypeStruct((B, D), jnp.float32)] * 3,
    scratch_types=[
        pltpu.VMEM((BPW,), jnp.int32),
        pltpu.VMEM((BPW,), jnp.int32),
        pltpu.VMEM((BPW,), jnp.int32),
        pltpu.SemaphoreType.DMA,
    ],
    compiler_params=pltpu.CompilerParams(use_tc_tiling_on_sc=True),
)(_sc_gather3)


BLK = 2048


def _tc_linear_body(u_ref, p_ref, n_ref, wu_ref, bu_ref, wi_ref, bi_ref,
                    uo_ref, po_ref, no_ref):
    wu = wu_ref[...]
    wi = wi_ref[...]
    dn = (((1,), (1,)), ((), ()))
    uo_ref[...] = lax.dot_general(u_ref[...], wu, dn,
                                  preferred_element_type=jnp.float32) + bu_ref[...]
    po_ref[...] = lax.dot_general(p_ref[...], wi, dn,
                                  preferred_element_type=jnp.float32) + bi_ref[...]
    no_ref[...] = lax.dot_general(n_ref[...], wi, dn,
                                  preferred_element_type=jnp.float32) + bi_ref[...]


def _tc_linear(U, P, N, Wu, bu2, Wi, bi2):
    row_spec = pl.BlockSpec((BLK, D), lambda i: (i, 0))
    w_spec = pl.BlockSpec((D, D), lambda i: (0, 0))
    b_spec = pl.BlockSpec((1, D), lambda i: (0, 0))
    return pl.pallas_call(
        _tc_linear_body,
        grid=(B // BLK,),
        in_specs=[row_spec, row_spec, row_spec, w_spec, b_spec, w_spec, b_spec],
        out_specs=[row_spec, row_spec, row_spec],
        out_shape=[jax.ShapeDtypeStruct((B, D), jnp.float32)] * 3,
    )(U, P, N, Wu, bu2, Wi, bi2)


def kernel(user_emb, item_emb, Wu, bu, Wi, bi, uid, iid, nid):
    uidr = uid.astype(jnp.int32).reshape(NW, BPW)
    iidr = iid.astype(jnp.int32).reshape(NW, BPW)
    nidr = nid.astype(jnp.int32).reshape(NW, BPW)
    U, P, N = _gather3(user_emb, item_emb, uidr, iidr, nidr)
    u_r, p_r, n_r = _tc_linear(U, P, N, Wu, bu.reshape(1, D), Wi, bi.reshape(1, D))
    return (u_r, p_r, n_r, P)


# speedup vs baseline: 3.0990x; 3.0990x over previous
import functools

import jax
import jax.numpy as jnp
from jax import lax
from jax.experimental import pallas as pl
from jax.experimental.pallas import tpu as pltpu
from jax.experimental.pallas import tpu_sc as plsc

B = 16384
D = 64
NC = 2
NS = 16
V = 1000000
HPW_U = B // NS
HPW_I = 2 * B // NS
PAD = 16
NT = (V + 127) // 128
LAST_T = NT - 1
NPREF = NT + 1 + 2 * PAD
RB = 64


def _scan_gather(table_hbm, sidx_hbm, spos_hbm, pref_hbm, out_ref, hpw, hbase,
                 sidx_l, spos_l, pref_l, tilebuf, rowbuf, ts0, ts1, wsem):
    pltpu.sync_copy(sidx_hbm.at[pl.ds(hbase, hpw + PAD)],
                    sidx_l.at[pl.ds(0, hpw + PAD)])
    pltpu.sync_copy(spos_hbm.at[pl.ds(hbase, hpw + PAD)],
                    spos_l.at[pl.ds(0, hpw + PAD)])
    pltpu.sync_copy(pref_hbm, pref_l)
    t0 = lax.shift_right_logical(sidx_l[pl.ds(0, 16)][0], 7)
    t1 = lax.shift_right_logical(sidx_l[pl.ds(hpw - 16, 16)][15], 7)
    iota16 = lax.iota(jnp.int32, 16)

    dsrc = out_ref.at[pl.ds(0, 1), :]
    ddst = rowbuf.at[pl.ds(RB, 1), :]
    for _ in range(RB):
        pltpu.async_copy(dsrc, ddst, wsem)

    def fetch(t, b):
        off = pl.multiple_of(t * 128, 128)
        sem = ts0 if b == 0 else ts1
        pltpu.async_copy(table_hbm.at[:, pl.ds(off, 128)], tilebuf.at[b], sem)

    def wait_tile(b):
        sem = ts0 if b == 0 else ts1
        pltpu.make_async_copy(table_hbm.at[:, pl.ds(0, 128)],
                              tilebuf.at[b], sem).wait()

    fetch(t0, 0)
    fetch(jnp.minimum(t0 + 1, t1), 1)
    npairs = lax.div(t1 - t0 + 2, 2)

    def hit_factory(t, bvec):
        def hit_body(k, iss):
            r = sidx_l[pl.ds(k, 16)][0]
            pos = spos_l[pl.ds(k, 16)][0]
            col = r - t * 128
            slot = lax.rem(iss, RB)
            pltpu.make_async_copy(out_ref.at[pl.ds(0, 1), :],
                                  rowbuf.at[pl.ds(RB, 1), :], wsem).wait()
            colv = jnp.full((16,), col, jnp.int32)
            rowslot = rowbuf.at[slot]
            for q in range(4):
                v4 = plsc.load_gather(tilebuf, [bvec, iota16 + 16 * q, colv])
                rowslot[pl.ds(16 * q, 16)] = v4
            pltpu.async_copy(rowbuf.at[pl.ds(slot, 1), :],
                             out_ref.at[pl.ds(pos, 1), :], wsem)
            return iss + 1

        return hit_body

    def pair_body(j, iss):
        for b in (0, 1):
            t = t0 + 2 * j + b
            wait_tile(b)
            h0 = pref_l[pl.ds(t, 16)][0] - hbase
            h1 = pref_l[pl.ds(t + 1, 16)][0] - hbase
            h0 = jnp.minimum(jnp.maximum(h0, 0), hpw)
            h1 = jnp.minimum(jnp.maximum(h1, 0), hpw)
            bvec = jnp.full((16,), b, jnp.int32)
            iss = lax.fori_loop(h0, h1, hit_factory(t, bvec), iss)
            fetch(jnp.minimum(t + 2, t1), b)
        return iss

    lax.fori_loop(0, npairs, pair_body, jnp.int32(0))
    wait_tile(0)
    wait_tile(1)
    for _ in range(RB):
        pltpu.make_async_copy(out_ref.at[pl.ds(0, 1), :],
                              rowbuf.at[pl.ds(RB, 1), :], wsem).wait()


def _sc_body(user_hbm, item_hbm, su_hbm, ou_hbm, pu_hbm, si_hbm, oi_hbm,
             pi_hbm, u_out, pn_out,
             sidx_l, spos_l, pref_l, tilebuf, rowbuf, ts0, ts1, wsem):
    wid = lax.axis_index("c") * NS + lax.axis_index("s")
    k = lax.rem(wid, NS)

    @pl.when(wid < NS)
    def _():
        _scan_gather(user_hbm, su_hbm, ou_hbm, pu_hbm, u_out, HPW_U,
                     k * HPW_U, sidx_l, spos_l, pref_l, tilebuf, rowbuf,
                     ts0, ts1, wsem)

    @pl.when(wid >= NS)
    def _():
        _scan_gather(item_hbm, si_hbm, oi_hbm, pi_hbm, pn_out, HPW_I,
                     k * HPW_I, sidx_l, spos_l, pref_l, tilebuf, rowbuf,
                     ts0, ts1, wsem)


_gather3 = functools.partial(
    pl.kernel,
    mesh=plsc.VectorSubcoreMesh(core_axis_name="c", subcore_axis_name="s"),
    out_type=[jax.ShapeDtypeStruct((B, D), jnp.float32),
              jax.ShapeDtypeStruct((2 * B, D), jnp.float32)],
    scratch_types=[
        pltpu.VMEM((HPW_I + PAD,), jnp.int32),
        pltpu.VMEM((HPW_I + PAD,), jnp.int32),
        pltpu.VMEM((NPREF,), jnp.int32),
        pltpu.VMEM((2, D, 128), jnp.float32),
        pltpu.VMEM((RB + 1, D), jnp.float32),
        pltpu.SemaphoreType.DMA,
        pltpu.SemaphoreType.DMA,
        pltpu.SemaphoreType.DMA,
    ],
    compiler_params=pltpu.CompilerParams(use_tc_tiling_on_sc=True,
                                         needs_layout_passes=False),
)(_sc_body)


BLK = 2048


def _tc_linear_body(u_ref, p_ref, n_ref, wu_ref, bu_ref, wi_ref, bi_ref,
                    uo_ref, po_ref, no_ref, pc_ref):
    wu = wu_ref[...]
    wi = wi_ref[...]
    dn = (((1,), (1,)), ((), ()))
    p = p_ref[...]
    uo_ref[...] = lax.dot_general(u_ref[...], wu, dn,
                                  preferred_element_type=jnp.float32) + bu_ref[...]
    po_ref[...] = lax.dot_general(p, wi, dn,
                                  preferred_element_type=jnp.float32) + bi_ref[...]
    no_ref[...] = lax.dot_general(n_ref[...], wi, dn,
                                  preferred_element_type=jnp.float32) + bi_ref[...]
    pc_ref[...] = p


def _tc_linear(U, PN, Wu, bu2, Wi, bi2):
    row_spec = pl.BlockSpec((BLK, D), lambda i: (i, 0))
    n_spec = pl.BlockSpec((BLK, D), lambda i: (i + B // BLK, 0))
    w_spec = pl.BlockSpec((D, D), lambda i: (0, 0))
    b_spec = pl.BlockSpec((1, D), lambda i: (0, 0))
    return pl.pallas_call(
        _tc_linear_body,
        grid=(B // BLK,),
        in_specs=[row_spec, row_spec, n_spec, w_spec, b_spec, w_spec, b_spec],
        out_specs=[row_spec, row_spec, row_spec, row_spec],
        out_shape=[jax.ShapeDtypeStruct((B, D), jnp.float32)] * 4,
    )(U, PN, PN, Wu, bu2, Wi, bi2)


def _prefix(sorted_vals, n):
    counts = jnp.bincount(lax.shift_right_logical(sorted_vals, 7), length=NT)
    pref = jnp.concatenate([
        jnp.zeros((1,), jnp.int32),
        jnp.cumsum(counts, dtype=jnp.int32),
        jnp.full((2 * PAD,), n, jnp.int32),
    ])
    return pref


def kernel(user_emb, item_emb, Wu, bu, Wi, bi, uid, iid, nid):
    i32max = jnp.full((PAD,), 2**31 - 1, jnp.int32)
    izero = jnp.zeros((PAD,), jnp.int32)
    su, ou = lax.sort((uid.astype(jnp.int32),
                       jnp.arange(B, dtype=jnp.int32)), num_keys=1)
    comb = jnp.concatenate([iid.astype(jnp.int32), nid.astype(jnp.int32)])
    si, oi = lax.sort((comb, jnp.arange(2 * B, dtype=jnp.int32)), num_keys=1)
    pu = _prefix(su, B)
    pi = _prefix(si, 2 * B)
    su_p = jnp.concatenate([su, i32max])
    ou_p = jnp.concatenate([ou, izero])
    si_p = jnp.concatenate([si, i32max])
    oi_p = jnp.concatenate([oi, izero])
    U, PN = _gather3(user_emb.T, item_emb.T, su_p, ou_p, pu, si_p, oi_p, pi)
    u_r, p_r, n_r, P = _tc_linear(U, PN, Wu, bu.reshape(1, D), Wi,
                                  bi.reshape(1, D))
    return (u_r, p_r, n_r, P)


# speedup vs baseline: 4.2016x; 1.3558x over previous
import functools

import jax
import jax.numpy as jnp
from jax import lax
from jax.experimental import pallas as pl
from jax.experimental.pallas import tpu as pltpu
from jax.experimental.pallas import tpu_sc as plsc

B = 16384
D = 64
NC = 2
NS = 16
V = 1000000
HPW_U = B // NS
HPW_I = 2 * B // NS
PAD = 16
NT = (V + 127) // 128
LAST_T = NT - 1
NPREF = NT + 1 + 2 * PAD
RB = 64
NBUF = 4


def _scan_gather(table_hbm, sidx_hbm, spos_hbm, pref_hbm, out_ref, hpw, hbase,
                 sidx_l, spos_l, pref_l, tilebuf, rowbuf, tsems, wsem):
    pltpu.sync_copy(sidx_hbm.at[pl.ds(hbase, hpw + PAD)],
                    sidx_l.at[pl.ds(0, hpw + PAD)])
    pltpu.sync_copy(spos_hbm.at[pl.ds(hbase, hpw + PAD)],
                    spos_l.at[pl.ds(0, hpw + PAD)])
    pltpu.sync_copy(pref_hbm, pref_l)
    t0 = lax.shift_right_logical(sidx_l[pl.ds(0, 16)][0], 7)
    t1 = lax.shift_right_logical(sidx_l[pl.ds(hpw - 16, 16)][15], 7)
    iota16 = lax.iota(jnp.int32, 16)

    dsrc = out_ref.at[pl.ds(0, 1), :]
    ddst = rowbuf.at[pl.ds(RB, 1), :]
    for _ in range(RB):
        pltpu.async_copy(dsrc, ddst, wsem)

    def fetch(t, b):
        off = pl.multiple_of(t * 128, 128)
        sem = tsems[b]
        pltpu.async_copy(table_hbm.at[:, pl.ds(off, 128)], tilebuf.at[b], sem)

    def wait_tile(b):
        sem = tsems[b]
        pltpu.make_async_copy(table_hbm.at[:, pl.ds(0, 128)],
                              tilebuf.at[b], sem).wait()

    for b in range(NBUF):
        fetch(jnp.minimum(t0 + b, t1), b)
    npairs = lax.div(t1 - t0 + NBUF, NBUF)

    def hit_factory(t, bvec):
        def hit_body(k, iss):
            r = sidx_l[pl.ds(k, 16)][0]
            pos = spos_l[pl.ds(k, 16)][0]
            col = r - t * 128
            slot = lax.rem(iss, RB)
            pltpu.make_async_copy(out_ref.at[pl.ds(0, 1), :],
                                  rowbuf.at[pl.ds(RB, 1), :], wsem).wait()
            colv = jnp.full((16,), col, jnp.int32)
            rowslot = rowbuf.at[slot]
            for q in range(4):
                v4 = plsc.load_gather(tilebuf, [bvec, iota16 + 16 * q, colv])
                rowslot[pl.ds(16 * q, 16)] = v4
            pltpu.async_copy(rowbuf.at[pl.ds(slot, 1), :],
                             out_ref.at[pl.ds(pos, 1), :], wsem)
            return iss + 1

        return hit_body

    def pair_body(j, iss):
        for b in range(NBUF):
            t = t0 + NBUF * j + b
            wait_tile(b)
            h0 = pref_l[pl.ds(t, 16)][0] - hbase
            h1 = pref_l[pl.ds(t + 1, 16)][0] - hbase
            h0 = jnp.minimum(jnp.maximum(h0, 0), hpw)
            h1 = jnp.minimum(jnp.maximum(h1, 0), hpw)
            bvec = jnp.full((16,), b, jnp.int32)
            iss = lax.fori_loop(h0, h1, hit_factory(t, bvec), iss)
            fetch(jnp.minimum(t + NBUF, t1), b)
        return iss

    lax.fori_loop(0, npairs, pair_body, jnp.int32(0))
    for b in range(NBUF):
        wait_tile(b)
    for _ in range(RB):
        pltpu.make_async_copy(out_ref.at[pl.ds(0, 1), :],
                              rowbuf.at[pl.ds(RB, 1), :], wsem).wait()


def _sc_body(user_hbm, item_hbm, su_hbm, ou_hbm, pu_hbm, si_hbm, oi_hbm,
             pi_hbm, u_out, pn_out,
             sidx_l, spos_l, pref_l, tilebuf, rowbuf, ts0, ts1, ts2, ts3,
             wsem):
    wid = lax.axis_index("c") * NS + lax.axis_index("s")
    k = lax.rem(wid, NS)

    @pl.when(wid < NS)
    def _():
        _scan_gather(user_hbm, su_hbm, ou_hbm, pu_hbm, u_out, HPW_U,
                     k * HPW_U, sidx_l, spos_l, pref_l, tilebuf, rowbuf,
                     (ts0, ts1, ts2, ts3), wsem)

    @pl.when(wid >= NS)
    def _():
        _scan_gather(item_hbm, si_hbm, oi_hbm, pi_hbm, pn_out, HPW_I,
                     k * HPW_I, sidx_l, spos_l, pref_l, tilebuf, rowbuf,
                     (ts0, ts1, ts2, ts3), wsem)


_gather3 = functools.partial(
    pl.kernel,
    mesh=plsc.VectorSubcoreMesh(core_axis_name="c", subcore_axis_name="s"),
    out_type=[jax.ShapeDtypeStruct((B, D), jnp.float32),
              jax.ShapeDtypeStruct((2 * B, D), jnp.float32)],
    scratch_types=[
        pltpu.VMEM((HPW_I + PAD,), jnp.int32),
        pltpu.VMEM((HPW_I + PAD,), jnp.int32),
        pltpu.VMEM((NPREF,), jnp.int32),
        pltpu.VMEM((NBUF, D, 128), jnp.float32),
        pltpu.VMEM((RB + 1, D), jnp.float32),
        pltpu.SemaphoreType.DMA,
        pltpu.SemaphoreType.DMA,
        pltpu.SemaphoreType.DMA,
        pltpu.SemaphoreType.DMA,
        pltpu.SemaphoreType.DMA,
    ],
    compiler_params=pltpu.CompilerParams(use_tc_tiling_on_sc=True,
                                         needs_layout_passes=False),
)(_sc_body)


BLK = 2048


def _tc_linear_body(u_ref, p_ref, n_ref, wu_ref, bu_ref, wi_ref, bi_ref,
                    uo_ref, po_ref, no_ref, pc_ref):
    wu = wu_ref[...]
    wi = wi_ref[...]
    dn = (((1,), (1,)), ((), ()))
    p = p_ref[...]
    uo_ref[...] = lax.dot_general(u_ref[...], wu, dn,
                                  preferred_element_type=jnp.float32) + bu_ref[...]
    po_ref[...] = lax.dot_general(p, wi, dn,
                                  preferred_element_type=jnp.float32) + bi_ref[...]
    no_ref[...] = lax.dot_general(n_ref[...], wi, dn,
                                  preferred_element_type=jnp.float32) + bi_ref[...]
    pc_ref[...] = p


def _tc_linear(U, PN, Wu, bu2, Wi, bi2):
    row_spec = pl.BlockSpec((BLK, D), lambda i: (i, 0))
    n_spec = pl.BlockSpec((BLK, D), lambda i: (i + B // BLK, 0))
    w_spec = pl.BlockSpec((D, D), lambda i: (0, 0))
    b_spec = pl.BlockSpec((1, D), lambda i: (0, 0))
    return pl.pallas_call(
        _tc_linear_body,
        grid=(B // BLK,),
        in_specs=[row_spec, row_spec, n_spec, w_spec, b_spec, w_spec, b_spec],
        out_specs=[row_spec, row_spec, row_spec, row_spec],
        out_shape=[jax.ShapeDtypeStruct((B, D), jnp.float32)] * 4,
    )(U, PN, PN, Wu, bu2, Wi, bi2)


def _prefix(sorted_vals, n):
    counts = jnp.bincount(lax.shift_right_logical(sorted_vals, 7), length=NT)
    pref = jnp.concatenate([
        jnp.zeros((1,), jnp.int32),
        jnp.cumsum(counts, dtype=jnp.int32),
        jnp.full((2 * PAD,), n, jnp.int32),
    ])
    return pref


def kernel(user_emb, item_emb, Wu, bu, Wi, bi, uid, iid, nid):
    i32max = jnp.full((PAD,), 2**31 - 1, jnp.int32)
    izero = jnp.zeros((PAD,), jnp.int32)
    su, ou = lax.sort((uid.astype(jnp.int32),
                       jnp.arange(B, dtype=jnp.int32)), num_keys=1)
    comb = jnp.concatenate([iid.astype(jnp.int32), nid.astype(jnp.int32)])
    si, oi = lax.sort((comb, jnp.arange(2 * B, dtype=jnp.int32)), num_keys=1)
    pu = _prefix(su, B)
    pi = _prefix(si, 2 * B)
    su_p = jnp.concatenate([su, i32max])
    ou_p = jnp.concatenate([ou, izero])
    si_p = jnp.concatenate([si, i32max])
    oi_p = jnp.concatenate([oi, izero])
    U, PN = _gather3(user_emb.T, item_emb.T, su_p, ou_p, pu, si_p, oi_p, pi)
    u_r, p_r, n_r, P = _tc_linear(U, PN, Wu, bu.reshape(1, D), Wi,
                                  bi.reshape(1, D))
    return (u_r, p_r, n_r, P)


# speedup vs baseline: 4.4963x; 1.0701x over previous
import functools

import jax
import jax.numpy as jnp
from jax import lax
from jax.experimental import pallas as pl
from jax.experimental.pallas import tpu as pltpu
from jax.experimental.pallas import tpu_sc as plsc

B = 16384
D = 64
NC = 2
NS = 16
V = 1000000
HPW_U = B // NS
HPW_I = 2 * B // NS
PAD = 16
NT = (V + 127) // 128
LAST_T = NT - 1
NPREF = NT + 1 + 2 * PAD
RB = 64
NBUF = 8


def _scan_gather(table_hbm, sidx_hbm, spos_hbm, pref_hbm, out_ref, hpw, hbase,
                 sidx_l, spos_l, pref_l, tilebuf, rowbuf, tsems, wsem):
    pltpu.sync_copy(sidx_hbm.at[pl.ds(hbase, hpw + PAD)],
                    sidx_l.at[pl.ds(0, hpw + PAD)])
    pltpu.sync_copy(spos_hbm.at[pl.ds(hbase, hpw + PAD)],
                    spos_l.at[pl.ds(0, hpw + PAD)])
    pltpu.sync_copy(pref_hbm, pref_l)
    t0 = lax.shift_right_logical(sidx_l[pl.ds(0, 16)][0], 7)
    t1 = lax.shift_right_logical(sidx_l[pl.ds(hpw - 16, 16)][15], 7)
    iota16 = lax.iota(jnp.int32, 16)

    dsrc = out_ref.at[pl.ds(0, 1), :]
    ddst = rowbuf.at[pl.ds(RB, 1), :]
    for _ in range(RB):
        pltpu.async_copy(dsrc, ddst, wsem)

    def fetch(t, b):
        off = pl.multiple_of(t * 128, 128)
        sem = tsems[b]
        pltpu.async_copy(table_hbm.at[:, pl.ds(off, 128)], tilebuf.at[b], sem)

    def wait_tile(b):
        sem = tsems[b]
        pltpu.make_async_copy(table_hbm.at[:, pl.ds(0, 128)],
                              tilebuf.at[b], sem).wait()

    for b in range(NBUF):
        fetch(jnp.minimum(t0 + b, t1), b)
    npairs = lax.div(t1 - t0 + NBUF, NBUF)

    def hit_factory(t, bvec):
        def hit_body(k, iss):
            r = sidx_l[pl.ds(k, 16)][0]
            pos = spos_l[pl.ds(k, 16)][0]
            col = r - t * 128
            slot = lax.rem(iss, RB)
            pltpu.make_async_copy(out_ref.at[pl.ds(0, 1), :],
                                  rowbuf.at[pl.ds(RB, 1), :], wsem).wait()
            colv = jnp.full((16,), col, jnp.int32)
            rowslot = rowbuf.at[slot]
            for q in range(4):
                v4 = plsc.load_gather(tilebuf, [bvec, iota16 + 16 * q, colv])
                rowslot[pl.ds(16 * q, 16)] = v4
            pltpu.async_copy(rowbuf.at[pl.ds(slot, 1), :],
                             out_ref.at[pl.ds(pos, 1), :], wsem)
            return iss + 1

        return hit_body

    def pair_body(j, iss):
        for b in range(NBUF):
            t = t0 + NBUF * j + b
            wait_tile(b)
            h0 = pref_l[pl.ds(t, 16)][0] - hbase
            h1 = pref_l[pl.ds(t + 1, 16)][0] - hbase
            h0 = jnp.minimum(jnp.maximum(h0, 0), hpw)
            h1 = jnp.minimum(jnp.maximum(h1, 0), hpw)
            bvec = jnp.full((16,), b, jnp.int32)
            iss = lax.fori_loop(h0, h1, hit_factory(t, bvec), iss)
            fetch(jnp.minimum(t + NBUF, t1), b)
        return iss

    lax.fori_loop(0, npairs, pair_body, jnp.int32(0))
    for b in range(NBUF):
        wait_tile(b)
    for _ in range(RB):
        pltpu.make_async_copy(out_ref.at[pl.ds(0, 1), :],
                              rowbuf.at[pl.ds(RB, 1), :], wsem).wait()


def _sc_body(user_hbm, item_hbm, su_hbm, ou_hbm, pu_hbm, si_hbm, oi_hbm,
             pi_hbm, u_out, pn_out,
             sidx_l, spos_l, pref_l, tilebuf, rowbuf, ts0, ts1, ts2, ts3,
             ts4, ts5, ts6, ts7, wsem):
    wid = lax.axis_index("c") * NS + lax.axis_index("s")
    k = lax.rem(wid, NS)

    @pl.when(wid < NS)
    def _():
        _scan_gather(user_hbm, su_hbm, ou_hbm, pu_hbm, u_out, HPW_U,
                     k * HPW_U, sidx_l, spos_l, pref_l, tilebuf, rowbuf,
                     (ts0, ts1, ts2, ts3, ts4, ts5, ts6, ts7), wsem)

    @pl.when(wid >= NS)
    def _():
        _scan_gather(item_hbm, si_hbm, oi_hbm, pi_hbm, pn_out, HPW_I,
                     k * HPW_I, sidx_l, spos_l, pref_l, tilebuf, rowbuf,
                     (ts0, ts1, ts2, ts3, ts4, ts5, ts6, ts7), wsem)


_gather3 = functools.partial(
    pl.kernel,
    mesh=plsc.VectorSubcoreMesh(core_axis_name="c", subcore_axis_name="s"),
    out_type=[jax.ShapeDtypeStruct((B, D), jnp.float32),
              jax.ShapeDtypeStruct((2 * B, D), jnp.float32)],
    scratch_types=[
        pltpu.VMEM((HPW_I + PAD,), jnp.int32),
        pltpu.VMEM((HPW_I + PAD,), jnp.int32),
        pltpu.VMEM((NPREF,), jnp.int32),
        pltpu.VMEM((NBUF, D, 128), jnp.float32),
        pltpu.VMEM((RB + 1, D), jnp.float32),
        pltpu.SemaphoreType.DMA,
        pltpu.SemaphoreType.DMA,
        pltpu.SemaphoreType.DMA,
        pltpu.SemaphoreType.DMA,
        pltpu.SemaphoreType.DMA,
        pltpu.SemaphoreType.DMA,
        pltpu.SemaphoreType.DMA,
        pltpu.SemaphoreType.DMA,
        pltpu.SemaphoreType.DMA,
    ],
    compiler_params=pltpu.CompilerParams(use_tc_tiling_on_sc=True,
                                         needs_layout_passes=False),
)(_sc_body)


BLK = 2048


def _tc_linear_body(u_ref, p_ref, n_ref, wu_ref, bu_ref, wi_ref, bi_ref,
                    uo_ref, po_ref, no_ref, pc_ref):
    wu = wu_ref[...]
    wi = wi_ref[...]
    dn = (((1,), (1,)), ((), ()))
    p = p_ref[...]
    uo_ref[...] = lax.dot_general(u_ref[...], wu, dn,
                                  preferred_element_type=jnp.float32) + bu_ref[...]
    po_ref[...] = lax.dot_general(p, wi, dn,
                                  preferred_element_type=jnp.float32) + bi_ref[...]
    no_ref[...] = lax.dot_general(n_ref[...], wi, dn,
                                  preferred_element_type=jnp.float32) + bi_ref[...]
    pc_ref[...] = p


def _tc_linear(U, PN, Wu, bu2, Wi, bi2):
    row_spec = pl.BlockSpec((BLK, D), lambda i: (i, 0))
    n_spec = pl.BlockSpec((BLK, D), lambda i: (i + B // BLK, 0))
    w_spec = pl.BlockSpec((D, D), lambda i: (0, 0))
    b_spec = pl.BlockSpec((1, D), lambda i: (0, 0))
    return pl.pallas_call(
        _tc_linear_body,
        grid=(B // BLK,),
        in_specs=[row_spec, row_spec, n_spec, w_spec, b_spec, w_spec, b_spec],
        out_specs=[row_spec, row_spec, row_spec, row_spec],
        out_shape=[jax.ShapeDtypeStruct((B, D), jnp.float32)] * 4,
    )(U, PN, PN, Wu, bu2, Wi, bi2)


def _prefix(sorted_vals, n):
    counts = jnp.bincount(lax.shift_right_logical(sorted_vals, 7), length=NT)
    pref = jnp.concatenate([
        jnp.zeros((1,), jnp.int32),
        jnp.cumsum(counts, dtype=jnp.int32),
        jnp.full((2 * PAD,), n, jnp.int32),
    ])
    return pref


def kernel(user_emb, item_emb, Wu, bu, Wi, bi, uid, iid, nid):
    i32max = jnp.full((PAD,), 2**31 - 1, jnp.int32)
    izero = jnp.zeros((PAD,), jnp.int32)
    su, ou = lax.sort((uid.astype(jnp.int32),
                       jnp.arange(B, dtype=jnp.int32)), num_keys=1)
    comb = jnp.concatenate([iid.astype(jnp.int32), nid.astype(jnp.int32)])
    si, oi = lax.sort((comb, jnp.arange(2 * B, dtype=jnp.int32)), num_keys=1)
    pu = _prefix(su, B)
    pi = _prefix(si, 2 * B)
    su_p = jnp.concatenate([su, i32max])
    ou_p = jnp.concatenate([ou, izero])
    si_p = jnp.concatenate([si, i32max])
    oi_p = jnp.concatenate([oi, izero])
    U, PN = _gather3(user_emb.T, item_emb.T, su_p, ou_p, pu, si_p, oi_p, pi)
    u_r, p_r, n_r, P = _tc_linear(U, PN, Wu, bu.reshape(1, D), Wi,
                                  bi.reshape(1, D))
    return (u_r, p_r, n_r, P)


# speedup vs baseline: 4.9771x; 1.1070x over previous
import functools

import jax
import jax.numpy as jnp
from jax import lax
from jax.experimental import pallas as pl
from jax.experimental.pallas import tpu as pltpu
from jax.experimental.pallas import tpu_sc as plsc

B = 16384
D = 64
NC = 2
NS = 16
V = 1000000
HPW_U = B // NS
HPW_I = 2 * B // NS
PAD = 16
NT = (V + 127) // 128
LAST_T = NT - 1
NPREF = NT + 1 + 2 * PAD
RB = 64
NBUF = 8


def _scan_gather(table_hbm, sidx_hbm, spos_hbm, pref_hbm, out_ref, hpw, hbase,
                 sidx_l, spos_l, pref_l, tilebuf, rowbuf, tsems, wsem):
    pltpu.sync_copy(sidx_hbm.at[pl.ds(hbase, hpw + PAD)],
                    sidx_l.at[pl.ds(0, hpw + PAD)])
    pltpu.sync_copy(spos_hbm.at[pl.ds(hbase, hpw + PAD)],
                    spos_l.at[pl.ds(0, hpw + PAD)])
    pltpu.sync_copy(pref_hbm, pref_l)
    t0 = lax.shift_right_logical(sidx_l[pl.ds(0, 16)][0], 7)
    t1 = lax.shift_right_logical(sidx_l[pl.ds(hpw - 16, 16)][15], 7)
    iota16 = lax.iota(jnp.int32, 16)

    dsrc = out_ref.at[pl.ds(0, 1), :]
    ddst = rowbuf.at[pl.ds(RB, 1), :]
    for _ in range(RB):
        pltpu.async_copy(dsrc, ddst, wsem)

    def fetch(t, b):
        off = pl.multiple_of(t * 128, 128)
        sem = tsems[b]
        pltpu.async_copy(table_hbm.at[:, pl.ds(off, 128)], tilebuf.at[b], sem)

    def wait_tile(b):
        sem = tsems[b]
        pltpu.make_async_copy(table_hbm.at[:, pl.ds(0, 128)],
                              tilebuf.at[b], sem).wait()

    for b in range(NBUF):
        fetch(jnp.minimum(t0 + b, t1), b)
    npairs = lax.div(t1 - t0 + NBUF, NBUF)

    def hit_factory(t, bvec):
        def hit_body(k, iss):
            r = sidx_l[pl.ds(k, 16)][0]
            pos = spos_l[pl.ds(k, 16)][0]
            col = r - t * 128
            slot = lax.rem(iss, RB)
            pltpu.make_async_copy(out_ref.at[pl.ds(0, 1), :],
                                  rowbuf.at[pl.ds(RB, 1), :], wsem).wait()
            colv = jnp.full((16,), col, jnp.int32)
            rowslot = rowbuf.at[slot]
            for q in range(4):
                v4 = plsc.load_gather(tilebuf, [bvec, iota16 + 16 * q, colv])
                rowslot[pl.ds(16 * q, 16)] = v4
            pltpu.async_copy(rowbuf.at[pl.ds(slot, 1), :],
                             out_ref.at[pl.ds(pos, 1), :], wsem)
            return iss + 1

        return hit_body

    def pair_body(j, iss):
        for b in range(NBUF):
            t = t0 + NBUF * j + b
            wait_tile(b)
            h0 = pref_l[pl.ds(t, 16)][0] - hbase
            h1 = pref_l[pl.ds(t + 1, 16)][0] - hbase
            h0 = jnp.minimum(jnp.maximum(h0, 0), hpw)
            h1 = jnp.minimum(jnp.maximum(h1, 0), hpw)
            bvec = jnp.full((16,), b, jnp.int32)
            iss = lax.fori_loop(h0, h1, hit_factory(t, bvec), iss)
            fetch(jnp.minimum(t + NBUF, t1), b)
        return iss

    lax.fori_loop(0, npairs, pair_body, jnp.int32(0))
    for b in range(NBUF):
        wait_tile(b)
    for _ in range(RB):
        pltpu.make_async_copy(out_ref.at[pl.ds(0, 1), :],
                              rowbuf.at[pl.ds(RB, 1), :], wsem).wait()


def _sc_body(user_hbm, item_hbm, su_hbm, ou_hbm, pu_hbm, si_hbm, oi_hbm,
             pi_hbm, u_out, pn_out,
             sidx_l, spos_l, pref_l, tilebuf, rowbuf, ts0, ts1, ts2, ts3,
             ts4, ts5, ts6, ts7, wsem):
    wid = lax.axis_index("c") * NS + lax.axis_index("s")
    k = lax.rem(wid, NS)

    @pl.when(wid < NS)
    def _():
        _scan_gather(user_hbm, su_hbm, ou_hbm, pu_hbm, u_out, HPW_U,
                     k * HPW_U, sidx_l, spos_l, pref_l, tilebuf, rowbuf,
                     (ts0, ts1, ts2, ts3, ts4, ts5, ts6, ts7), wsem)

    @pl.when(wid >= NS)
    def _():
        _scan_gather(item_hbm, si_hbm, oi_hbm, pi_hbm, pn_out, HPW_I,
                     k * HPW_I, sidx_l, spos_l, pref_l, tilebuf, rowbuf,
                     (ts0, ts1, ts2, ts3, ts4, ts5, ts6, ts7), wsem)


_gather3 = functools.partial(
    pl.kernel,
    mesh=plsc.VectorSubcoreMesh(core_axis_name="c", subcore_axis_name="s"),
    out_type=[jax.ShapeDtypeStruct((B, D), jnp.float32),
              jax.ShapeDtypeStruct((2 * B, D), jnp.float32)],
    scratch_types=[
        pltpu.VMEM((HPW_I + PAD,), jnp.int32),
        pltpu.VMEM((HPW_I + PAD,), jnp.int32),
        pltpu.VMEM((NPREF,), jnp.int32),
        pltpu.VMEM((NBUF, D, 128), jnp.float32),
        pltpu.VMEM((RB + 1, D), jnp.float32),
        pltpu.SemaphoreType.DMA,
        pltpu.SemaphoreType.DMA,
        pltpu.SemaphoreType.DMA,
        pltpu.SemaphoreType.DMA,
        pltpu.SemaphoreType.DMA,
        pltpu.SemaphoreType.DMA,
        pltpu.SemaphoreType.DMA,
        pltpu.SemaphoreType.DMA,
        pltpu.SemaphoreType.DMA,
    ],
    compiler_params=pltpu.CompilerParams(use_tc_tiling_on_sc=True,
                                         needs_layout_passes=False),
)(_sc_body)


BLK = 2048


def _tc_linear_body(u_ref, p_ref, n_ref, wu_ref, bu_ref, wi_ref, bi_ref,
                    uo_ref, po_ref, no_ref, pc_ref):
    wu = wu_ref[...]
    wi = wi_ref[...]
    dn = (((1,), (1,)), ((), ()))
    p = p_ref[...]
    uo_ref[...] = lax.dot_general(wu, u_ref[...], dn,
                                  preferred_element_type=jnp.float32) + bu_ref[...]
    po_ref[...] = lax.dot_general(wi, p, dn,
                                  preferred_element_type=jnp.float32) + bi_ref[...]
    no_ref[...] = lax.dot_general(wi, n_ref[...], dn,
                                  preferred_element_type=jnp.float32) + bi_ref[...]
    pc_ref[...] = p.T


def _tc_linear(U, PN, Wu, bu2, Wi, bi2):
    row_spec = pl.BlockSpec((BLK, D), lambda i: (i, 0))
    n_spec = pl.BlockSpec((BLK, D), lambda i: (i + B // BLK, 0))
    w_spec = pl.BlockSpec((D, D), lambda i: (0, 0))
    b_spec = pl.BlockSpec((D, 1), lambda i: (0, 0))
    t_spec = pl.BlockSpec((D, BLK), lambda i: (0, i))
    return pl.pallas_call(
        _tc_linear_body,
        grid=(B // BLK,),
        in_specs=[row_spec, row_spec, n_spec, w_spec, b_spec, w_spec, b_spec],
        out_specs=[t_spec, t_spec, t_spec, t_spec],
        out_shape=[jax.ShapeDtypeStruct((D, B), jnp.float32)] * 4,
    )(U, PN, PN, Wu, bu2, Wi, bi2)


def _prefix(sorted_vals, n):
    counts = jnp.bincount(lax.shift_right_logical(sorted_vals, 7), length=NT)
    pref = jnp.concatenate([
        jnp.zeros((1,), jnp.int32),
        jnp.cumsum(counts, dtype=jnp.int32),
        jnp.full((2 * PAD,), n, jnp.int32),
    ])
    return pref


def kernel(user_emb, item_emb, Wu, bu, Wi, bi, uid, iid, nid):
    i32max = jnp.full((PAD,), 2**31 - 1, jnp.int32)
    izero = jnp.zeros((PAD,), jnp.int32)
    su, ou = lax.sort((uid.astype(jnp.int32),
                       jnp.arange(B, dtype=jnp.int32)), num_keys=1)
    comb = jnp.concatenate([iid.astype(jnp.int32), nid.astype(jnp.int32)])
    si, oi = lax.sort((comb, jnp.arange(2 * B, dtype=jnp.int32)), num_keys=1)
    pu = _prefix(su, B)
    pi = _prefix(si, 2 * B)
    su_p = jnp.concatenate([su, i32max])
    ou_p = jnp.concatenate([ou, izero])
    si_p = jnp.concatenate([si, i32max])
    oi_p = jnp.concatenate([oi, izero])
    U, PN = _gather3(user_emb.T, item_emb.T, su_p, ou_p, pu, si_p, oi_p, pi)
    u_t, p_t, n_t, P_t = _tc_linear(U, PN, Wu, bu.reshape(D, 1), Wi,
                                    bi.reshape(D, 1))
    return (u_t.T, p_t.T, n_t.T, P_t.T)
